# baseline (speedup 1.0000x reference)
import functools

import jax
import jax.numpy as jnp
from jax import lax
from jax.experimental import pallas as pl
from jax.experimental.pallas import tpu as pltpu
from jax.experimental.pallas import tpu_sc as plsc

NUM_CHUNKS = 320000
NUM_CELLS = 10000
DIM = 128

NC = 2
NS = 16
NW = NC * NS

CPW = NUM_CHUNKS // NW
IDXW = 80
NITER = CPW // IDXW
NCP = 10240
WB = NCP // NS
NVEC = IDXW // 16
NBUF = 3


def _sc_partials(chunk_features, ids1):
    mesh = plsc.VectorSubcoreMesh(core_axis_name="c", subcore_axis_name="s")

    @functools.partial(
        pl.kernel,
        out_type=(
            jax.ShapeDtypeStruct((NC, NCP, DIM), jnp.float32),
            jax.ShapeDtypeStruct((NW, 1, NCP), jnp.float32),
        ),
        mesh=mesh,
        compiler_params=pltpu.CompilerParams(needs_layout_passes=False),
        scratch_types=[
            pltpu.VMEM((NBUF, IDXW, DIM), jnp.float32),
            pltpu.VMEM((NBUF, IDXW), jnp.int32),
            pltpu.VMEM((NCP,), jnp.float32),
            pltpu.VMEM_SHARED((NCP, DIM), jnp.float32),
            pltpu.SemaphoreType.DMA((NBUF,)),
            pltpu.SemaphoreType.DMA((NBUF,)),
            pltpu.SemaphoreType.DMA((NBUF,)),
            pltpu.SemaphoreType.DMA,
        ],
    )
    def k(cf_hbm, ids_hbm,
          outs_hbm, outc_hbm, rows_v, idb_v, cnt_v, acc_sh,
          sem_rows, sem_ids, sem_sc, sem_z):
        c = lax.axis_index("c")
        s = lax.axis_index("s")
        wid = s * NC + c
        base = wid * CPW

        def start_in(i, b):
            pltpu.async_copy(cf_hbm.at[pl.ds(base + i * IDXW, IDXW)],
                             rows_v.at[b], sem_rows.at[b])
            pltpu.async_copy(ids_hbm.at[pl.ds(base + i * IDXW, IDXW)],
                             idb_v.at[b], sem_ids.at[b])

        def wait_in(i, b):
            pltpu.make_async_copy(cf_hbm.at[pl.ds(base + i * IDXW, IDXW)],
                                  rows_v.at[b], sem_rows.at[b]).wait()
            pltpu.make_async_copy(ids_hbm.at[pl.ds(base + i * IDXW, IDXW)],
                                  idb_v.at[b], sem_ids.at[b]).wait()

        zeros16 = jnp.zeros((16,), jnp.float32)

        @pl.loop(0, IDXW)
        def _(j):
            for kk in range(DIM // 16):
                rows_v[2, j, pl.ds(kk * 16, 16)] = zeros16

        for p in range(WB // IDXW):
            pltpu.async_copy(rows_v.at[2],
                             acc_sh.at[pl.ds(s * WB + p * IDXW, IDXW)],
                             sem_z)
        start_in(0, 0)
        start_in(1, 1)

        @pl.loop(0, NCP // 16)
        def _(j):
            cnt_v[pl.ds(j * 16, 16)] = zeros16

        for p in range(WB // IDXW):
            pltpu.make_async_copy(rows_v.at[2],
                                  acc_sh.at[pl.ds(s * WB + p * IDXW, IDXW)],
                                  sem_z).wait()
        plsc.subcore_barrier()

        ones16 = jnp.ones((16,), jnp.float32)

        def wait_sc(b):
            pltpu.make_async_copy(rows_v.at[b], acc_sh.at[idb_v.at[b]],
                                  sem_sc.at[b]).wait()

        @pl.loop(0, NITER)
        def _(i):
            b = lax.rem(i, NBUF)

            wait_in(i, b)
            pltpu.async_copy(rows_v.at[b], acc_sh.at[idb_v.at[b]],
                             sem_sc.at[b], add=True)
            for kk in range(NVEC):
                idv = idb_v[b, pl.ds(kk * 16, 16)]
                plsc.addupdate_scatter(cnt_v, [idv], ones16)

            @pl.when(i >= 1)
            def _():
                wait_sc(lax.rem(i - 1, NBUF))

            @pl.when(i + 2 < NITER)
            def _():
                start_in(i + 2, lax.rem(i + 2, NBUF))

        wait_sc((NITER - 1) % NBUF)
        plsc.subcore_barrier()

        pltpu.sync_copy(acc_sh.at[pl.ds(s * WB, WB)],
                        outs_hbm.at[c, pl.ds(s * WB, WB)])
        pltpu.sync_copy(cnt_v, outc_hbm.at[wid, 0])

    return k(chunk_features, ids1)


def _merge_body(s_ref, c_ref, o_ref):
    sums = s_ref[0] + s_ref[1]
    cnt = jnp.sum(c_ref[:, 0, :], axis=0)
    o_ref[...] = sums / jnp.maximum(cnt, 1.0)[:, None]


_MERGE_ROWS = 2048


def _merge(sums, counts):
    grid = (NUM_CELLS + _MERGE_ROWS - 1) // _MERGE_ROWS
    return pl.pallas_call(
        _merge_body,
        grid=(grid,),
        in_specs=[
            pl.BlockSpec((NC, _MERGE_ROWS, DIM), lambda i: (0, i, 0)),
            pl.BlockSpec((NW, 1, _MERGE_ROWS), lambda i: (0, 0, i)),
        ],
        out_specs=pl.BlockSpec((_MERGE_ROWS, DIM), lambda i: (i, 0)),
        out_shape=jax.ShapeDtypeStruct((NUM_CELLS, DIM), jnp.float32),
    )(sums, counts)


def kernel(chunk_features, segment_ids):
    ids1 = segment_ids.astype(jnp.int32)
    sums, counts = _sc_partials(chunk_features, ids1)
    return _merge(sums, counts)


# speedup vs baseline: 1.0269x; 1.0269x over previous
import functools

import jax
import jax.numpy as jnp
from jax import lax
from jax.experimental import pallas as pl
from jax.experimental.pallas import tpu as pltpu
from jax.experimental.pallas import tpu_sc as plsc

NUM_CHUNKS = 320000
NUM_CELLS = 10000
DIM = 128

NC = 2
NS = 16
NW = NC * NS

CPW = NUM_CHUNKS // NW
IDXW = 80
NITER = CPW // IDXW
NCP = 10240
WB = NCP // NS
NVEC = IDXW // 16
NBUF = 3


def _sc_partials(chunk_features, ids1):
    mesh = plsc.VectorSubcoreMesh(core_axis_name="c", subcore_axis_name="s")

    @functools.partial(
        pl.kernel,
        out_type=(
            jax.ShapeDtypeStruct((NC, NCP, DIM), jnp.float32),
            jax.ShapeDtypeStruct((NW, 1, NCP), jnp.float32),
        ),
        mesh=mesh,
        compiler_params=pltpu.CompilerParams(needs_layout_passes=False),
        scratch_types=[
            pltpu.VMEM((NBUF, IDXW, DIM), jnp.float32),
            pltpu.VMEM((NBUF, IDXW), jnp.int32),
            pltpu.VMEM((NCP,), jnp.float32),
            pltpu.VMEM_SHARED((NCP, DIM), jnp.float32),
            pltpu.SemaphoreType.DMA((NBUF,)),
            pltpu.SemaphoreType.DMA((NBUF,)),
            pltpu.SemaphoreType.DMA((NBUF,)),
            pltpu.SemaphoreType.DMA,
        ],
    )
    def k(cf_hbm, ids_hbm,
          outs_hbm, outc_hbm, rows_v, idb_v, cnt_v, acc_sh,
          sem_rows, sem_ids, sem_sc, sem_z):
        c = lax.axis_index("c")
        s = lax.axis_index("s")
        wid = s * NC + c
        base = wid * CPW

        def start_in(i, b):
            pltpu.async_copy(cf_hbm.at[pl.ds(base + i * IDXW, IDXW)],
                             rows_v.at[b], sem_rows.at[b])
            pltpu.async_copy(ids_hbm.at[pl.ds(base + i * IDXW, IDXW)],
                             idb_v.at[b], sem_ids.at[b])

        def wait_in(i, b):
            pltpu.make_async_copy(cf_hbm.at[pl.ds(base + i * IDXW, IDXW)],
                                  rows_v.at[b], sem_rows.at[b]).wait()
            pltpu.make_async_copy(ids_hbm.at[pl.ds(base + i * IDXW, IDXW)],
                                  idb_v.at[b], sem_ids.at[b]).wait()

        zeros16 = jnp.zeros((16,), jnp.float32)

        @pl.loop(0, IDXW)
        def _(j):
            for kk in range(DIM // 16):
                rows_v[2, j, pl.ds(kk * 16, 16)] = zeros16

        for p in range(WB // IDXW):
            pltpu.async_copy(rows_v.at[2],
                             acc_sh.at[pl.ds(s * WB + p * IDXW, IDXW)],
                             sem_z)
        start_in(0, 0)
        start_in(1, 1)

        @pl.loop(0, NCP // 16)
        def _(j):
            cnt_v[pl.ds(j * 16, 16)] = zeros16

        for p in range(WB // IDXW):
            pltpu.make_async_copy(rows_v.at[2],
                                  acc_sh.at[pl.ds(s * WB + p * IDXW, IDXW)],
                                  sem_z).wait()
        plsc.subcore_barrier()

        ones16 = jnp.ones((16,), jnp.float32)

        def wait_sc(b):
            pltpu.make_async_copy(rows_v.at[b], acc_sh.at[idb_v.at[b]],
                                  sem_sc.at[b]).wait()

        @pl.loop(0, NITER)
        def _(i):
            b = lax.rem(i, NBUF)

            @pl.when(i >= 1)
            def _():
                wait_sc(lax.rem(i - 1, NBUF))

            @pl.when(i + 2 < NITER)
            def _():
                start_in(i + 2, lax.rem(i + 2, NBUF))

            wait_in(i, b)
            pltpu.async_copy(rows_v.at[b], acc_sh.at[idb_v.at[b]],
                             sem_sc.at[b], add=True)
            for kk in range(NVEC):
                idv = idb_v[b, pl.ds(kk * 16, 16)]
                plsc.addupdate_scatter(cnt_v, [idv], ones16)

        wait_sc((NITER - 1) % NBUF)
        plsc.subcore_barrier()

        pltpu.sync_copy(acc_sh.at[pl.ds(s * WB, WB)],
                        outs_hbm.at[c, pl.ds(s * WB, WB)])
        pltpu.sync_copy(cnt_v, outc_hbm.at[wid, 0])

    return k(chunk_features, ids1)


def _merge_body(s_ref, c_ref, o_ref):
    sums = s_ref[0] + s_ref[1]
    cnt = jnp.sum(c_ref[:, 0, :], axis=0)
    o_ref[...] = sums / jnp.maximum(cnt, 1.0)[:, None]


_MERGE_ROWS = 2048


def _merge(sums, counts):
    grid = (NUM_CELLS + _MERGE_ROWS - 1) // _MERGE_ROWS
    return pl.pallas_call(
        _merge_body,
        grid=(grid,),
        in_specs=[
            pl.BlockSpec((NC, _MERGE_ROWS, DIM), lambda i: (0, i, 0)),
            pl.BlockSpec((NW, 1, _MERGE_ROWS), lambda i: (0, 0, i)),
        ],
        out_specs=pl.BlockSpec((_MERGE_ROWS, DIM), lambda i: (i, 0)),
        out_shape=jax.ShapeDtypeStruct((NUM_CELLS, DIM), jnp.float32),
    )(sums, counts)


def kernel(chunk_features, segment_ids):
    ids1 = segment_ids.astype(jnp.int32)
    sums, counts = _sc_partials(chunk_features, ids1)
    return _merge(sums, counts)
